# Initial kernel scaffold; baseline (speedup 1.0000x reference)
import jax
import jax.numpy as jnp
from jax import lax
from jax.experimental import pallas as pl
from jax.experimental.pallas import tpu as pltpu
from jax.experimental.pallas import tpu_sc as plsc

NC = 2
NS = 16
CHUNK = 80


def _degree_kernel(n_pad, rows_per_tile):
    mesh = plsc.VectorSubcoreMesh(core_axis_name="c", subcore_axis_name="s")
    zchunk = n_pad // NS

    def body(src_hbm, dst_hbm, dsp_hbm, ddp_hbm, sidx, didx, ones_v, zbuf, dsh, ddh):
        c = lax.axis_index("c")
        s = lax.axis_index("s")

        def fill_zero(i, _):
            zbuf[pl.ds(i * 16, 16)] = jnp.zeros((16,), jnp.float32)
            return ()
        lax.fori_loop(0, zchunk // 16, fill_zero, ())

        def fill_one(i, _):
            ones_v[pl.ds(i * 16, 16)] = jnp.ones((16,), jnp.float32)
            return ()
        lax.fori_loop(0, CHUNK // 16, fill_one, ())

        pltpu.sync_copy(zbuf, dsh.at[pl.ds(s * zchunk, zchunk)])
        pltpu.sync_copy(zbuf, ddh.at[pl.ds(s * zchunk, zchunk)])
        plsc.subcore_barrier()

        row0 = (c * NS + s) * rows_per_tile
        pltpu.sync_copy(src_hbm.at[pl.ds(row0, rows_per_tile)], sidx)
        pltpu.sync_copy(dst_hbm.at[pl.ds(row0, rows_per_tile)], didx)

        def step(r, _):
            pltpu.sync_copy(ones_v, dsh.at[sidx.at[r]], add=True)
            pltpu.sync_copy(ones_v, ddh.at[didx.at[r]], add=True)
            return ()
        lax.fori_loop(0, rows_per_tile, step, ())
        plsc.subcore_barrier()

        pltpu.sync_copy(dsh.at[pl.ds(s * zchunk, zchunk)],
                        dsp_hbm.at[c, pl.ds(s * zchunk, zchunk)])
        pltpu.sync_copy(ddh.at[pl.ds(s * zchunk, zchunk)],
                        ddp_hbm.at[c, pl.ds(s * zchunk, zchunk)])

    return pl.kernel(
        body,
        out_type=(
            jax.ShapeDtypeStruct((NC, n_pad), jnp.float32),
            jax.ShapeDtypeStruct((NC, n_pad), jnp.float32),
        ),
        mesh=mesh,
        scratch_types=[
            pltpu.VMEM((rows_per_tile, CHUNK), jnp.int32),
            pltpu.VMEM((rows_per_tile, CHUNK), jnp.int32),
            pltpu.VMEM((CHUNK,), jnp.float32),
            pltpu.VMEM((zchunk,), jnp.float32),
            pltpu.VMEM_SHARED((n_pad,), jnp.float32),
            pltpu.VMEM_SHARED((n_pad,), jnp.float32),
        ],
    )


def _aggregate_kernel(n_nodes, d, rows_per_tile):
    mesh = plsc.VectorSubcoreMesh(core_axis_name="c", subcore_axis_name="s")
    out_rows = n_nodes // NS
    zrows_n = out_rows // 5

    def body(h_hbm, src_hbm, dst_hbm, out_hbm, sidx, didx, buf0, buf1, zrows,
             acc, gsem0, gsem1, ssem0, ssem1):
        c = lax.axis_index("c")
        s = lax.axis_index("s")
        dg = d // 16

        def fill_zero(i, _):
            zrows[i // dg, pl.ds((i % dg) * 16, 16)] = jnp.zeros((16,), jnp.float32)
            return ()
        lax.fori_loop(0, zrows_n * dg, fill_zero, ())
        for k in range(out_rows // zrows_n):
            pltpu.sync_copy(zrows, acc.at[pl.ds(s * out_rows + k * zrows_n, zrows_n)])
        plsc.subcore_barrier()

        row0 = (c * NS + s) * rows_per_tile
        pltpu.sync_copy(src_hbm.at[pl.ds(row0, rows_per_tile)], sidx)
        pltpu.sync_copy(dst_hbm.at[pl.ds(row0, rows_per_tile)], didx)

        def step(k, _):
            r0 = 2 * k
            r1 = 2 * k + 1
            g0 = pltpu.async_copy(h_hbm.at[sidx.at[r0]], buf0, gsem0)
            g1 = pltpu.async_copy(h_hbm.at[sidx.at[r1]], buf1, gsem1)
            g0.wait()
            s0 = pltpu.async_copy(buf0, acc.at[didx.at[r0]], ssem0, add=True)
            g1.wait()
            s1 = pltpu.async_copy(buf1, acc.at[didx.at[r1]], ssem1, add=True)
            s0.wait()
            s1.wait()
            return ()
        lax.fori_loop(0, rows_per_tile // 2, step, ())
        if rows_per_tile % 2:
            r = rows_per_tile - 1
            pltpu.async_copy(h_hbm.at[sidx.at[r]], buf0, gsem0).wait()
            pltpu.sync_copy(buf0, acc.at[didx.at[r]], add=True)
        plsc.subcore_barrier()

        pltpu.sync_copy(acc.at[pl.ds(s * out_rows, out_rows)],
                        out_hbm.at[c, pl.ds(s * out_rows, out_rows)])

    return pl.kernel(
        body,
        out_type=jax.ShapeDtypeStruct((NC, n_nodes, d), jnp.float32),
        mesh=mesh,
        scratch_types=[
            pltpu.VMEM((rows_per_tile, CHUNK), jnp.int32),
            pltpu.VMEM((rows_per_tile, CHUNK), jnp.int32),
            pltpu.VMEM((CHUNK, d), jnp.float32),
            pltpu.VMEM((CHUNK, d), jnp.float32),
            pltpu.VMEM((zrows_n, d), jnp.float32),
            pltpu.VMEM_SHARED((n_nodes, d), jnp.float32),
            pltpu.SemaphoreType.DMA,
            pltpu.SemaphoreType.DMA,
            pltpu.SemaphoreType.DMA,
            pltpu.SemaphoreType.DMA,
        ],
    )


def _transform_body(x_ref, w_ref, b_ref, ds0_ref, ds1_ref, dd0_ref, dd1_ref,
                    h_ref, dd_ref):
    h = jnp.dot(x_ref[...], w_ref[...], preferred_element_type=jnp.float32)
    h = h + b_ref[...]
    dinv_src = lax.rsqrt(ds0_ref[...] + ds1_ref[...] + 1e-05)
    h_ref[...] = h * dinv_src
    dd_ref[...] = lax.rsqrt(dd0_ref[...] + dd1_ref[...] + 1e-05)


def _combine_body(p0_ref, p1_ref, dd_ref, out_ref):
    out_ref[...] = (p0_ref[...] + p1_ref[...]) * dd_ref[...]


@jax.jit
def kernel(x, edge_index, W, b):
    n, d_in = x.shape
    d_out = W.shape[1]
    e = edge_index.shape[1]
    assert e % (CHUNK * NC * NS) == 0 and n % NS == 0

    n_pad = ((n + NS * 16 - 1) // (NS * 16)) * (NS * 16)
    rows = e // CHUNK
    rows_per_tile = rows // (NC * NS)

    src2d = edge_index[0].astype(jnp.int32).reshape(rows, CHUNK)
    dst2d = edge_index[1].astype(jnp.int32).reshape(rows, CHUNK)

    dsp, ddp = _degree_kernel(n_pad, rows_per_tile)(src2d, dst2d)
    ds0 = dsp[0, :n, None]
    ds1 = dsp[1, :n, None]
    dd0 = ddp[0, :n, None]
    dd1 = ddp[1, :n, None]

    blk = 1000
    grid = (n // blk,)
    h_scaled, dinv_dst = pl.pallas_call(
        _transform_body,
        grid=grid,
        in_specs=[
            pl.BlockSpec((blk, d_in), lambda i: (i, 0)),
            pl.BlockSpec((d_in, d_out), lambda i: (0, 0)),
            pl.BlockSpec((1, d_out), lambda i: (0, 0)),
            pl.BlockSpec((blk, 1), lambda i: (i, 0)),
            pl.BlockSpec((blk, 1), lambda i: (i, 0)),
            pl.BlockSpec((blk, 1), lambda i: (i, 0)),
            pl.BlockSpec((blk, 1), lambda i: (i, 0)),
        ],
        out_specs=[
            pl.BlockSpec((blk, d_out), lambda i: (i, 0)),
            pl.BlockSpec((blk, 1), lambda i: (i, 0)),
        ],
        out_shape=[
            jax.ShapeDtypeStruct((n, d_out), jnp.float32),
            jax.ShapeDtypeStruct((n, 1), jnp.float32),
        ],
    )(x, W, b.reshape(1, d_out), ds0, ds1, dd0, dd1)

    partials = _aggregate_kernel(n, d_out, rows_per_tile)(h_scaled, src2d, dst2d)

    out = pl.pallas_call(
        _combine_body,
        grid=grid,
        in_specs=[
            pl.BlockSpec((blk, d_out), lambda i: (i, 0)),
            pl.BlockSpec((blk, d_out), lambda i: (i, 0)),
            pl.BlockSpec((blk, 1), lambda i: (i, 0)),
        ],
        out_specs=pl.BlockSpec((blk, d_out), lambda i: (i, 0)),
        out_shape=jax.ShapeDtypeStruct((n, d_out), jnp.float32),
    )(partials[0], partials[1], dinv_dst)
    return out


# speedup vs baseline: 24.5451x; 24.5451x over previous
import jax
import jax.numpy as jnp
from jax import lax
from jax.experimental import pallas as pl
from jax.experimental.pallas import tpu as pltpu
from jax.experimental.pallas import tpu_sc as plsc

NC = 2
NS = 16
CHUNK = 80
BROWS = 25
KBLK = 5


def _degree_kernel(n_pad):
    mesh = plsc.VectorSubcoreMesh(core_axis_name="c", subcore_axis_name="s")
    zchunk = n_pad // NS

    def body(src_hbm, dst_hbm, dsp_hbm, ddp_hbm, sidx, didx, ones_v, zbuf, dsh, ddh):
        c = lax.axis_index("c")
        s = lax.axis_index("s")

        def fill_zero(i, _):
            zbuf[pl.ds(i * 16, 16)] = jnp.zeros((16,), jnp.float32)
            return ()
        lax.fori_loop(0, zchunk // 16, fill_zero, ())

        def fill_one(i, _):
            ones_v[pl.ds(i * 16, 16)] = jnp.ones((16,), jnp.float32)
            return ()
        lax.fori_loop(0, CHUNK // 16, fill_one, ())

        pltpu.sync_copy(zbuf, dsh.at[pl.ds(s * zchunk, zchunk)])
        pltpu.sync_copy(zbuf, ddh.at[pl.ds(s * zchunk, zchunk)])
        plsc.subcore_barrier()

        w = c * NS + s

        def block(k, _):
            pltpu.sync_copy(src_hbm.at[w, k], sidx)
            pltpu.sync_copy(dst_hbm.at[w, k], didx)

            def step(r, _):
                pltpu.sync_copy(ones_v, dsh.at[sidx.at[r]], add=True)
                pltpu.sync_copy(ones_v, ddh.at[didx.at[r]], add=True)
                return ()
            lax.fori_loop(0, BROWS, step, ())
            return ()
        lax.fori_loop(0, KBLK, block, ())
        plsc.subcore_barrier()

        pltpu.sync_copy(dsh.at[pl.ds(s * zchunk, zchunk)], dsp_hbm.at[c, s, 0])
        pltpu.sync_copy(ddh.at[pl.ds(s * zchunk, zchunk)], ddp_hbm.at[c, s, 0])

    return pl.kernel(
        body,
        out_type=(
            jax.ShapeDtypeStruct((NC, NS, 1, zchunk), jnp.float32),
            jax.ShapeDtypeStruct((NC, NS, 1, zchunk), jnp.float32),
        ),
        mesh=mesh,
        scratch_types=[
            pltpu.VMEM((BROWS, CHUNK), jnp.int32),
            pltpu.VMEM((BROWS, CHUNK), jnp.int32),
            pltpu.VMEM((CHUNK,), jnp.float32),
            pltpu.VMEM((zchunk,), jnp.float32),
            pltpu.VMEM_SHARED((n_pad,), jnp.float32),
            pltpu.VMEM_SHARED((n_pad,), jnp.float32),
        ],
    )


def _aggregate_kernel(n_nodes, d):
    mesh = plsc.VectorSubcoreMesh(core_axis_name="c", subcore_axis_name="s")
    out_rows = n_nodes // NS
    zchunks = n_nodes // CHUNK

    def body(h_hbm, src_hbm, dst_hbm, out_hbm, sidx, didx, buf0, buf1,
             acc, gsem0, gsem1, ssem0, ssem1):
        c = lax.axis_index("c")
        s = lax.axis_index("s")
        dg = d // 16

        def fill_zero(i, _):
            buf0[i // dg, pl.ds((i % dg) * 16, 16)] = jnp.zeros((16,), jnp.float32)
            return ()
        lax.fori_loop(0, CHUNK * dg, fill_zero, ())

        def zstep(i, _):
            j = s + i * NS

            @pl.when(j < zchunks)
            def _():
                pltpu.sync_copy(buf0, acc.at[pl.ds(j * CHUNK, CHUNK)])
            return ()
        lax.fori_loop(0, (zchunks + NS - 1) // NS, zstep, ())
        plsc.subcore_barrier()

        w = c * NS + s

        def block(k, _):
            pltpu.sync_copy(src_hbm.at[w, k], sidx)
            pltpu.sync_copy(dst_hbm.at[w, k], didx)

            def step(j, _):
                r0 = 2 * j
                r1 = 2 * j + 1
                g0 = pltpu.async_copy(h_hbm.at[sidx.at[r0]], buf0, gsem0)
                g1 = pltpu.async_copy(h_hbm.at[sidx.at[r1]], buf1, gsem1)
                g0.wait()
                s0 = pltpu.async_copy(buf0, acc.at[didx.at[r0]], ssem0, add=True)
                g1.wait()
                s1 = pltpu.async_copy(buf1, acc.at[didx.at[r1]], ssem1, add=True)
                s0.wait()
                s1.wait()
                return ()
            lax.fori_loop(0, BROWS // 2, step, ())
            if BROWS % 2:
                r = BROWS - 1
                pltpu.async_copy(h_hbm.at[sidx.at[r]], buf0, gsem0).wait()
                pltpu.sync_copy(buf0, acc.at[didx.at[r]], add=True)
            return ()
        lax.fori_loop(0, KBLK, block, ())
        plsc.subcore_barrier()

        pltpu.sync_copy(acc.at[pl.ds(s * out_rows, out_rows)], out_hbm.at[c, s])

    return pl.kernel(
        body,
        out_type=jax.ShapeDtypeStruct((NC, NS, out_rows, d), jnp.float32),
        mesh=mesh,
        scratch_types=[
            pltpu.VMEM((BROWS, CHUNK), jnp.int32),
            pltpu.VMEM((BROWS, CHUNK), jnp.int32),
            pltpu.VMEM((CHUNK, d), jnp.float32),
            pltpu.VMEM((CHUNK, d), jnp.float32),
            pltpu.VMEM_SHARED((n_nodes, d), jnp.float32),
            pltpu.SemaphoreType.DMA,
            pltpu.SemaphoreType.DMA,
            pltpu.SemaphoreType.DMA,
            pltpu.SemaphoreType.DMA,
        ],
    )


def _transform_body(x_ref, w_ref, b_ref, ds0_ref, ds1_ref, dd0_ref, dd1_ref,
                    h_ref, dd_ref):
    h = jnp.dot(x_ref[...], w_ref[...], preferred_element_type=jnp.float32)
    h = h + b_ref[...]
    dinv_src = lax.rsqrt(ds0_ref[...] + ds1_ref[...] + 1e-05)
    h_ref[...] = h * dinv_src
    dd_ref[...] = lax.rsqrt(dd0_ref[...] + dd1_ref[...] + 1e-05)


def _combine_body(p0_ref, p1_ref, dd_ref, out_ref):
    out_ref[...] = (p0_ref[...] + p1_ref[...]) * dd_ref[...]


@jax.jit
def kernel(x, edge_index, W, b):
    n, d_in = x.shape
    d_out = W.shape[1]
    e = edge_index.shape[1]
    assert e == NC * NS * KBLK * BROWS * CHUNK
    assert n % NS == 0 and n % CHUNK == 0

    n_pad = ((n + NS * 16 - 1) // (NS * 16)) * (NS * 16)

    src4d = edge_index[0].astype(jnp.int32).reshape(NC * NS, KBLK, BROWS, CHUNK)
    dst4d = edge_index[1].astype(jnp.int32).reshape(NC * NS, KBLK, BROWS, CHUNK)

    dsp, ddp = _degree_kernel(n_pad)(src4d, dst4d)
    dsp = dsp.reshape(NC, n_pad)
    ddp = ddp.reshape(NC, n_pad)
    ds0 = dsp[0, :n, None]
    ds1 = dsp[1, :n, None]
    dd0 = ddp[0, :n, None]
    dd1 = ddp[1, :n, None]

    blk = 1000
    grid = (n // blk,)
    h_scaled, dinv_dst = pl.pallas_call(
        _transform_body,
        grid=grid,
        in_specs=[
            pl.BlockSpec((blk, d_in), lambda i: (i, 0)),
            pl.BlockSpec((d_in, d_out), lambda i: (0, 0)),
            pl.BlockSpec((1, d_out), lambda i: (0, 0)),
            pl.BlockSpec((blk, 1), lambda i: (i, 0)),
            pl.BlockSpec((blk, 1), lambda i: (i, 0)),
            pl.BlockSpec((blk, 1), lambda i: (i, 0)),
            pl.BlockSpec((blk, 1), lambda i: (i, 0)),
        ],
        out_specs=[
            pl.BlockSpec((blk, d_out), lambda i: (i, 0)),
            pl.BlockSpec((blk, 1), lambda i: (i, 0)),
        ],
        out_shape=[
            jax.ShapeDtypeStruct((n, d_out), jnp.float32),
            jax.ShapeDtypeStruct((n, 1), jnp.float32),
        ],
    )(x, W, b.reshape(1, d_out), ds0, ds1, dd0, dd1)

    partials = _aggregate_kernel(n, d_out)(h_scaled, src4d, dst4d)
    partials = partials.reshape(NC, n, d_out)

    out = pl.pallas_call(
        _combine_body,
        grid=grid,
        in_specs=[
            pl.BlockSpec((blk, d_out), lambda i: (i, 0)),
            pl.BlockSpec((blk, d_out), lambda i: (i, 0)),
            pl.BlockSpec((blk, 1), lambda i: (i, 0)),
        ],
        out_specs=pl.BlockSpec((blk, d_out), lambda i: (i, 0)),
        out_shape=jax.ShapeDtypeStruct((n, d_out), jnp.float32),
    )(partials[0], partials[1], dinv_dst)
    return out


# speedup vs baseline: 28.5216x; 1.1620x over previous
import jax
import jax.numpy as jnp
from jax import lax
from jax.experimental import pallas as pl
from jax.experimental.pallas import tpu as pltpu
from jax.experimental.pallas import tpu_sc as plsc

NC = 2
NS = 16
CHUNK = 80
BROWS = 25
KBLK = 5


def _degree_kernel(n_pad):
    mesh = plsc.VectorSubcoreMesh(core_axis_name="c", subcore_axis_name="s")
    zchunk = n_pad // NS

    def body(src_hbm, dst_hbm, dsp_hbm, ddp_hbm, sidx, didx, ones_v, zbuf, dsh, ddh):
        c = lax.axis_index("c")
        s = lax.axis_index("s")

        def fill_zero(i, _):
            zbuf[pl.ds(i * 16, 16)] = jnp.zeros((16,), jnp.float32)
            return ()
        lax.fori_loop(0, zchunk // 16, fill_zero, ())

        def fill_one(i, _):
            ones_v[pl.ds(i * 16, 16)] = jnp.ones((16,), jnp.float32)
            return ()
        lax.fori_loop(0, CHUNK // 16, fill_one, ())

        pltpu.sync_copy(zbuf, dsh.at[pl.ds(s * zchunk, zchunk)])
        pltpu.sync_copy(zbuf, ddh.at[pl.ds(s * zchunk, zchunk)])
        plsc.subcore_barrier()

        w = c * NS + s

        def block(k, _):
            pltpu.sync_copy(src_hbm.at[w, k], sidx)
            pltpu.sync_copy(dst_hbm.at[w, k], didx)

            def step(r, _):
                pltpu.sync_copy(ones_v, dsh.at[sidx.at[r]], add=True)
                pltpu.sync_copy(ones_v, ddh.at[didx.at[r]], add=True)
                return ()
            lax.fori_loop(0, BROWS, step, ())
            return ()
        lax.fori_loop(0, KBLK, block, ())
        plsc.subcore_barrier()

        pltpu.sync_copy(dsh.at[pl.ds(s * zchunk, zchunk)], dsp_hbm.at[c, s, 0])
        pltpu.sync_copy(ddh.at[pl.ds(s * zchunk, zchunk)], ddp_hbm.at[c, s, 0])

    return pl.kernel(
        body,
        out_type=(
            jax.ShapeDtypeStruct((NC, NS, 1, zchunk), jnp.float32),
            jax.ShapeDtypeStruct((NC, NS, 1, zchunk), jnp.float32),
        ),
        mesh=mesh,
        scratch_types=[
            pltpu.VMEM((BROWS, CHUNK), jnp.int32),
            pltpu.VMEM((BROWS, CHUNK), jnp.int32),
            pltpu.VMEM((CHUNK,), jnp.float32),
            pltpu.VMEM((zchunk,), jnp.float32),
            pltpu.VMEM_SHARED((n_pad,), jnp.float32),
            pltpu.VMEM_SHARED((n_pad,), jnp.float32),
        ],
    )


def _aggregate_kernel(n_nodes, d):
    mesh = plsc.VectorSubcoreMesh(core_axis_name="c", subcore_axis_name="s")
    out_rows = n_nodes // NS
    zchunks = n_nodes // CHUNK

    def body(h_hbm, src_hbm, dst_hbm, out_hbm, sidx, didx, bufs,
             acc, gsems, ssems):
        c = lax.axis_index("c")
        s = lax.axis_index("s")
        dg = d // 16

        def fill_zero(i, _):
            bufs[0, i // dg, pl.ds((i % dg) * 16, 16)] = jnp.zeros((16,), jnp.float32)
            return ()
        lax.fori_loop(0, CHUNK * dg, fill_zero, ())

        def zstep(i, _):
            j = s + i * NS

            @pl.when(j < zchunks)
            def _():
                pltpu.sync_copy(bufs.at[0], acc.at[pl.ds(j * CHUNK, CHUNK)])
            return ()
        lax.fori_loop(0, (zchunks + NS - 1) // NS, zstep, ())
        plsc.subcore_barrier()

        w = c * NS + s

        for k in range(KBLK):
            pltpu.sync_copy(src_hbm.at[w, k], sidx)
            pltpu.sync_copy(dst_hbm.at[w, k], didx)
            for r0 in range(2):
                pltpu.async_copy(h_hbm.at[sidx.at[r0]], bufs.at[r0], gsems.at[r0])

            def step(r, _):
                bi = r % 4
                pltpu.make_async_copy(
                    h_hbm.at[sidx.at[r]], bufs.at[bi], gsems.at[bi]).wait()
                pltpu.async_copy(bufs.at[bi], acc.at[didx.at[r]], ssems.at[bi],
                                 add=True)
                nr = r + 2

                @pl.when(nr < BROWS)
                def _():
                    nbi = nr % 4

                    @pl.when(nr >= 4)
                    def _():
                        pltpu.make_async_copy(
                            bufs.at[nbi], acc.at[didx.at[nr - 4]],
                            ssems.at[nbi]).wait()
                    pltpu.async_copy(h_hbm.at[sidx.at[nr]], bufs.at[nbi],
                                     gsems.at[nbi])
                return ()
            lax.fori_loop(0, BROWS, step, ())
            for rr in range(BROWS - 4, BROWS):
                bi = rr % 4
                pltpu.make_async_copy(
                    bufs.at[bi], acc.at[didx.at[rr]], ssems.at[bi]).wait()
        plsc.subcore_barrier()

        pltpu.sync_copy(acc.at[pl.ds(s * out_rows, out_rows)], out_hbm.at[c, s])

    return pl.kernel(
        body,
        out_type=jax.ShapeDtypeStruct((NC, NS, out_rows, d), jnp.float32),
        mesh=mesh,
        scratch_types=[
            pltpu.VMEM((BROWS, CHUNK), jnp.int32),
            pltpu.VMEM((BROWS, CHUNK), jnp.int32),
            pltpu.VMEM((4, CHUNK, d), jnp.float32),
            pltpu.VMEM_SHARED((n_nodes, d), jnp.float32),
            pltpu.SemaphoreType.DMA((4,)),
            pltpu.SemaphoreType.DMA((4,)),
        ],
    )


def _transform_body(x_ref, w_ref, b_ref, ds0_ref, ds1_ref, dd0_ref, dd1_ref,
                    h_ref, dd_ref):
    h = jnp.dot(x_ref[...], w_ref[...], preferred_element_type=jnp.float32)
    h = h + b_ref[...]
    dinv_src = lax.rsqrt(ds0_ref[...] + ds1_ref[...] + 1e-05)
    h_ref[...] = h * dinv_src
    dd_ref[...] = lax.rsqrt(dd0_ref[...] + dd1_ref[...] + 1e-05)


def _combine_body(p0_ref, p1_ref, dd_ref, out_ref):
    out_ref[...] = (p0_ref[...] + p1_ref[...]) * dd_ref[...]


@jax.jit
def kernel(x, edge_index, W, b):
    n, d_in = x.shape
    d_out = W.shape[1]
    e = edge_index.shape[1]
    assert e == NC * NS * KBLK * BROWS * CHUNK
    assert n % NS == 0 and n % CHUNK == 0

    n_pad = ((n + NS * 16 - 1) // (NS * 16)) * (NS * 16)

    src4d = edge_index[0].astype(jnp.int32).reshape(NC * NS, KBLK, BROWS, CHUNK)
    dst4d = edge_index[1].astype(jnp.int32).reshape(NC * NS, KBLK, BROWS, CHUNK)

    dsp, ddp = _degree_kernel(n_pad)(src4d, dst4d)
    dsp = dsp.reshape(NC, n_pad)
    ddp = ddp.reshape(NC, n_pad)
    ds0 = dsp[0, :n, None]
    ds1 = dsp[1, :n, None]
    dd0 = ddp[0, :n, None]
    dd1 = ddp[1, :n, None]

    blk = 1000
    grid = (n // blk,)
    h_scaled, dinv_dst = pl.pallas_call(
        _transform_body,
        grid=grid,
        in_specs=[
            pl.BlockSpec((blk, d_in), lambda i: (i, 0)),
            pl.BlockSpec((d_in, d_out), lambda i: (0, 0)),
            pl.BlockSpec((1, d_out), lambda i: (0, 0)),
            pl.BlockSpec((blk, 1), lambda i: (i, 0)),
            pl.BlockSpec((blk, 1), lambda i: (i, 0)),
            pl.BlockSpec((blk, 1), lambda i: (i, 0)),
            pl.BlockSpec((blk, 1), lambda i: (i, 0)),
        ],
        out_specs=[
            pl.BlockSpec((blk, d_out), lambda i: (i, 0)),
            pl.BlockSpec((blk, 1), lambda i: (i, 0)),
        ],
        out_shape=[
            jax.ShapeDtypeStruct((n, d_out), jnp.float32),
            jax.ShapeDtypeStruct((n, 1), jnp.float32),
        ],
    )(x, W, b.reshape(1, d_out), ds0, ds1, dd0, dd1)

    partials = _aggregate_kernel(n, d_out)(h_scaled, src4d, dst4d)
    partials = partials.reshape(NC, n, d_out)

    out = pl.pallas_call(
        _combine_body,
        grid=grid,
        in_specs=[
            pl.BlockSpec((blk, d_out), lambda i: (i, 0)),
            pl.BlockSpec((blk, d_out), lambda i: (i, 0)),
            pl.BlockSpec((blk, 1), lambda i: (i, 0)),
        ],
        out_specs=pl.BlockSpec((blk, d_out), lambda i: (i, 0)),
        out_shape=jax.ShapeDtypeStruct((n, d_out), jnp.float32),
    )(partials[0], partials[1], dinv_dst)
    return out


# speedup vs baseline: 34.1486x; 1.1973x over previous
import jax
import jax.numpy as jnp
from jax import lax
from jax.experimental import pallas as pl
from jax.experimental.pallas import tpu as pltpu
from jax.experimental.pallas import tpu_sc as plsc

NC = 2
NS = 16
CHUNK = 80
BROWS = 25
KBLK = 5


def _degree_kernel(n_pad):
    mesh = plsc.VectorSubcoreMesh(core_axis_name="c", subcore_axis_name="s")
    zchunk = n_pad // NS

    def body(edge_hbm, dsp_hbm, ddp_hbm, sidx, didx, ones_v, zbuf, dsh, ddh):
        c = lax.axis_index("c")
        s = lax.axis_index("s")

        def fill_zero(i, _):
            zbuf[pl.ds(i * 16, 16)] = jnp.zeros((16,), jnp.float32)
            return ()
        lax.fori_loop(0, zchunk // 16, fill_zero, ())

        def fill_one(i, _):
            ones_v[pl.ds(i * 16, 16)] = jnp.ones((16,), jnp.float32)
            return ()
        lax.fori_loop(0, CHUNK // 16, fill_one, ())

        pltpu.sync_copy(zbuf, dsh.at[pl.ds(s * zchunk, zchunk)])
        pltpu.sync_copy(zbuf, ddh.at[pl.ds(s * zchunk, zchunk)])
        plsc.subcore_barrier()

        w = c * NS + s

        def block(k, _):
            pltpu.sync_copy(edge_hbm.at[0, w, k], sidx)
            pltpu.sync_copy(edge_hbm.at[1, w, k], didx)

            def step(r, _):
                pltpu.sync_copy(ones_v, dsh.at[sidx.at[r]], add=True)
                pltpu.sync_copy(ones_v, ddh.at[didx.at[r]], add=True)
                return ()
            lax.fori_loop(0, BROWS, step, ())
            return ()
        lax.fori_loop(0, KBLK, block, ())
        plsc.subcore_barrier()

        pltpu.sync_copy(dsh.at[pl.ds(s * zchunk, zchunk)], dsp_hbm.at[c, s, 0])
        pltpu.sync_copy(ddh.at[pl.ds(s * zchunk, zchunk)], ddp_hbm.at[c, s, 0])

    return pl.kernel(
        body,
        out_type=(
            jax.ShapeDtypeStruct((NC, NS, 1, zchunk), jnp.float32),
            jax.ShapeDtypeStruct((NC, NS, 1, zchunk), jnp.float32),
        ),
        mesh=mesh,
        scratch_types=[
            pltpu.VMEM((BROWS, CHUNK), jnp.int32),
            pltpu.VMEM((BROWS, CHUNK), jnp.int32),
            pltpu.VMEM((CHUNK,), jnp.float32),
            pltpu.VMEM((zchunk,), jnp.float32),
            pltpu.VMEM_SHARED((n_pad,), jnp.float32),
            pltpu.VMEM_SHARED((n_pad,), jnp.float32),
        ],
    )


def _aggregate_kernel(n_nodes, d):
    mesh = plsc.VectorSubcoreMesh(core_axis_name="c", subcore_axis_name="s")
    zchunks = n_nodes // CHUNK

    def body(h_hbm, edge_hbm, out_hbm, sidx, didx, bufs, acc, gsems, ssems):
        c = lax.axis_index("c")
        s = lax.axis_index("s")
        dg = d // 16

        def fill_zero(i, _):
            bufs[0, i // dg, pl.ds((i % dg) * 16, 16)] = jnp.zeros((16,), jnp.float32)
            return ()
        lax.fori_loop(0, CHUNK * dg, fill_zero, ())

        def zstep(i, _):
            j = s + i * NS

            @pl.when(j < zchunks)
            def _():
                pltpu.sync_copy(bufs.at[0], acc.at[pl.ds(j * CHUNK, CHUNK)])
            return ()
        lax.fori_loop(0, (zchunks + NS - 1) // NS, zstep, ())
        plsc.subcore_barrier()

        w = c * NS + s

        for k in range(KBLK):
            pltpu.sync_copy(edge_hbm.at[0, w, k], sidx)
            pltpu.sync_copy(edge_hbm.at[1, w, k], didx)
            for r0 in range(2):
                pltpu.async_copy(h_hbm.at[sidx.at[r0]], bufs.at[r0], gsems.at[r0])

            def step(r, _):
                bi = r % 4
                pltpu.make_async_copy(
                    h_hbm.at[sidx.at[r]], bufs.at[bi], gsems.at[bi]).wait()
                pltpu.async_copy(bufs.at[bi], acc.at[didx.at[r]], ssems.at[bi],
                                 add=True)
                nr = r + 2

                @pl.when(nr < BROWS)
                def _():
                    nbi = nr % 4

                    @pl.when(nr >= 4)
                    def _():
                        pltpu.make_async_copy(
                            bufs.at[nbi], acc.at[didx.at[nr - 4]],
                            ssems.at[nbi]).wait()
                    pltpu.async_copy(h_hbm.at[sidx.at[nr]], bufs.at[nbi],
                                     gsems.at[nbi])
                return ()
            lax.fori_loop(0, BROWS, step, ())
            for rr in range(BROWS - 4, BROWS):
                bi = rr % 4
                pltpu.make_async_copy(
                    bufs.at[bi], acc.at[didx.at[rr]], ssems.at[bi]).wait()
        plsc.subcore_barrier()

        def wstep(i, _):
            j = s + i * NS

            @pl.when(j < zchunks)
            def _():
                pltpu.sync_copy(acc.at[pl.ds(j * CHUNK, CHUNK)],
                                out_hbm.at[c, pl.ds(j * CHUNK, CHUNK)])
            return ()
        lax.fori_loop(0, (zchunks + NS - 1) // NS, wstep, ())

    return pl.kernel(
        body,
        out_type=jax.ShapeDtypeStruct((NC, n_nodes, d), jnp.float32),
        mesh=mesh,
        scratch_types=[
            pltpu.VMEM((BROWS, CHUNK), jnp.int32),
            pltpu.VMEM((BROWS, CHUNK), jnp.int32),
            pltpu.VMEM((4, CHUNK, d), jnp.float32),
            pltpu.VMEM_SHARED((n_nodes, d), jnp.float32),
            pltpu.SemaphoreType.DMA((4,)),
            pltpu.SemaphoreType.DMA((4,)),
        ],
    )


def _matmul_body(x_ref, w_ref, b_ref, h_ref):
    h_ref[...] = jnp.dot(x_ref[...], w_ref[...],
                         preferred_element_type=jnp.float32) + b_ref[...]


def _scale_body(h_ref, ds0_ref, ds1_ref, dd0_ref, dd1_ref, hs_ref, dd_ref):
    row_s = ds0_ref[0, 0] + ds1_ref[0, 0]
    dinv_src = lax.rsqrt(jnp.transpose(row_s, (1, 0)) + 1e-05)
    hs_ref[...] = h_ref[...] * dinv_src
    row_d = dd0_ref[0, 0] + dd1_ref[0, 0]
    dd_ref[...] = lax.rsqrt(jnp.transpose(row_d, (1, 0)) + 1e-05)


def _combine_body(p0_ref, p1_ref, dd_ref, out_ref):
    out_ref[...] = (p0_ref[0] + p1_ref[0]) * dd_ref[...]


@jax.jit
def kernel(x, edge_index, W, b):
    n, d_in = x.shape
    d_out = W.shape[1]
    e = edge_index.shape[1]
    assert e == NC * NS * KBLK * BROWS * CHUNK
    assert n % NS == 0 and n % CHUNK == 0

    n_pad = ((n + NS * 16 - 1) // (NS * 16)) * (NS * 16)
    blk = n_pad // NS
    grid = (NS,)

    edge5d = edge_index.astype(jnp.int32).reshape(2, NC * NS, KBLK, BROWS, CHUNK)

    h = pl.pallas_call(
        _matmul_body,
        grid=grid,
        in_specs=[
            pl.BlockSpec((blk, d_in), lambda i: (i, 0)),
            pl.BlockSpec((d_in, d_out), lambda i: (0, 0)),
            pl.BlockSpec((1, d_out), lambda i: (0, 0)),
        ],
        out_specs=pl.BlockSpec((blk, d_out), lambda i: (i, 0)),
        out_shape=jax.ShapeDtypeStruct((n, d_out), jnp.float32),
    )(x, W, b.reshape(1, d_out))

    dsp, ddp = _degree_kernel(n_pad)(edge5d)

    h_scaled, dinv_dst = pl.pallas_call(
        _scale_body,
        grid=grid,
        in_specs=[
            pl.BlockSpec((blk, d_out), lambda i: (i, 0)),
            pl.BlockSpec((1, 1, 1, blk), lambda i: (0, i, 0, 0)),
            pl.BlockSpec((1, 1, 1, blk), lambda i: (1, i, 0, 0)),
            pl.BlockSpec((1, 1, 1, blk), lambda i: (0, i, 0, 0)),
            pl.BlockSpec((1, 1, 1, blk), lambda i: (1, i, 0, 0)),
        ],
        out_specs=[
            pl.BlockSpec((blk, d_out), lambda i: (i, 0)),
            pl.BlockSpec((blk, 1), lambda i: (i, 0)),
        ],
        out_shape=[
            jax.ShapeDtypeStruct((n, d_out), jnp.float32),
            jax.ShapeDtypeStruct((n_pad, 1), jnp.float32),
        ],
    )(h, dsp, dsp, ddp, ddp)

    partials = _aggregate_kernel(n, d_out)(h_scaled, edge5d)

    out = pl.pallas_call(
        _combine_body,
        grid=grid,
        in_specs=[
            pl.BlockSpec((1, blk, d_out), lambda i: (0, i, 0)),
            pl.BlockSpec((1, blk, d_out), lambda i: (1, i, 0)),
            pl.BlockSpec((blk, 1), lambda i: (i, 0)),
        ],
        out_specs=pl.BlockSpec((blk, d_out), lambda i: (i, 0)),
        out_shape=jax.ShapeDtypeStruct((n, d_out), jnp.float32),
    )(partials, partials, dinv_dst)
    return out


# speedup vs baseline: 35.0117x; 1.0253x over previous
import jax
import jax.numpy as jnp
from jax import lax
from jax.experimental import pallas as pl
from jax.experimental.pallas import tpu as pltpu
from jax.experimental.pallas import tpu_sc as plsc

NC = 2
NS = 16
CHUNK = 80
BROWS = 25
KBLK = 5


def _degree_kernel(n_pad):
    mesh = plsc.VectorSubcoreMesh(core_axis_name="c", subcore_axis_name="s")
    zchunk = n_pad // NS

    def body(edge_hbm, dsp_hbm, ddp_hbm, sidx, didx, ones_v, zbuf, dsh, ddh):
        c = lax.axis_index("c")
        s = lax.axis_index("s")

        def fill_zero(i, _):
            zbuf[pl.ds(i * 16, 16)] = jnp.zeros((16,), jnp.float32)
            return ()
        lax.fori_loop(0, zchunk // 16, fill_zero, ())

        def fill_one(i, _):
            ones_v[pl.ds(i * 16, 16)] = jnp.ones((16,), jnp.float32)
            return ()
        lax.fori_loop(0, CHUNK // 16, fill_one, ())

        pltpu.sync_copy(zbuf, dsh.at[pl.ds(s * zchunk, zchunk)])
        pltpu.sync_copy(zbuf, ddh.at[pl.ds(s * zchunk, zchunk)])
        plsc.subcore_barrier()

        w = c * NS + s

        def block(k, _):
            pltpu.sync_copy(edge_hbm.at[0, w, k], sidx)
            pltpu.sync_copy(edge_hbm.at[1, w, k], didx)

            def step(r, _):
                pltpu.sync_copy(ones_v, dsh.at[sidx.at[r]], add=True)
                pltpu.sync_copy(ones_v, ddh.at[didx.at[r]], add=True)
                return ()
            lax.fori_loop(0, BROWS, step, ())
            return ()
        lax.fori_loop(0, KBLK, block, ())
        plsc.subcore_barrier()

        pltpu.sync_copy(dsh.at[pl.ds(s * zchunk, zchunk)], dsp_hbm.at[c, s, 0])
        pltpu.sync_copy(ddh.at[pl.ds(s * zchunk, zchunk)], ddp_hbm.at[c, s, 0])

    return pl.kernel(
        body,
        out_type=(
            jax.ShapeDtypeStruct((NC, NS, 1, zchunk), jnp.float32),
            jax.ShapeDtypeStruct((NC, NS, 1, zchunk), jnp.float32),
        ),
        mesh=mesh,
        scratch_types=[
            pltpu.VMEM((BROWS, CHUNK), jnp.int32),
            pltpu.VMEM((BROWS, CHUNK), jnp.int32),
            pltpu.VMEM((CHUNK,), jnp.float32),
            pltpu.VMEM((zchunk,), jnp.float32),
            pltpu.VMEM_SHARED((n_pad,), jnp.float32),
            pltpu.VMEM_SHARED((n_pad,), jnp.float32),
        ],
    )


def _aggregate_kernel(n_nodes, d):
    mesh = plsc.VectorSubcoreMesh(core_axis_name="c", subcore_axis_name="s")
    zchunks = n_nodes // CHUNK

    def body(h_hbm, edge_hbm, out_hbm, sidx, didx, bufs, acc, gsems, ssems):
        c = lax.axis_index("c")
        s = lax.axis_index("s")
        dg = d // 16

        def fill_zero(i, _):
            bufs[0, i // dg, pl.ds((i % dg) * 16, 16)] = jnp.zeros((16,), jnp.float32)
            return ()
        lax.fori_loop(0, CHUNK * dg, fill_zero, ())

        def zstep(i, _):
            j = s + i * NS

            @pl.when(j < zchunks)
            def _():
                pltpu.sync_copy(bufs.at[0], acc.at[pl.ds(j * CHUNK, CHUNK)])
            return ()
        lax.fori_loop(0, (zchunks + NS - 1) // NS, zstep, ())
        plsc.subcore_barrier()

        w = c * NS + s

        for k in range(KBLK):
            pltpu.sync_copy(edge_hbm.at[0, w, k], sidx)
            pltpu.sync_copy(edge_hbm.at[1, w, k], didx)
            for r0 in range(2):
                pltpu.async_copy(h_hbm.at[sidx.at[r0]], bufs.at[r0], gsems.at[r0])

            for r in range(BROWS):
                bi = r % 4
                pltpu.make_async_copy(
                    h_hbm.at[sidx.at[r]], bufs.at[bi], gsems.at[bi]).wait()
                pltpu.async_copy(bufs.at[bi], acc.at[didx.at[r]], ssems.at[bi],
                                 add=True)
                nr = r + 2
                if nr < BROWS:
                    nbi = nr % 4
                    if nr >= 4:
                        pltpu.make_async_copy(
                            bufs.at[nbi], acc.at[didx.at[nr - 4]],
                            ssems.at[nbi]).wait()
                    pltpu.async_copy(h_hbm.at[sidx.at[nr]], bufs.at[nbi],
                                     gsems.at[nbi])
            for rr in range(BROWS - 4, BROWS):
                bi = rr % 4
                pltpu.make_async_copy(
                    bufs.at[bi], acc.at[didx.at[rr]], ssems.at[bi]).wait()
        plsc.subcore_barrier()

        def wstep(i, _):
            j = s + i * NS

            @pl.when(j < zchunks)
            def _():
                pltpu.sync_copy(acc.at[pl.ds(j * CHUNK, CHUNK)],
                                out_hbm.at[c, pl.ds(j * CHUNK, CHUNK)])
            return ()
        lax.fori_loop(0, (zchunks + NS - 1) // NS, wstep, ())

    return pl.kernel(
        body,
        out_type=jax.ShapeDtypeStruct((NC, n_nodes, d), jnp.float32),
        mesh=mesh,
        scratch_types=[
            pltpu.VMEM((BROWS, CHUNK), jnp.int32),
            pltpu.VMEM((BROWS, CHUNK), jnp.int32),
            pltpu.VMEM((4, CHUNK, d), jnp.float32),
            pltpu.VMEM_SHARED((n_nodes, d), jnp.float32),
            pltpu.SemaphoreType.DMA((4,)),
            pltpu.SemaphoreType.DMA((4,)),
        ],
    )


def _matmul_body(x_ref, w_ref, b_ref, h_ref):
    h_ref[...] = jnp.dot(x_ref[...], w_ref[...],
                         preferred_element_type=jnp.float32) + b_ref[...]


def _scale_body(h_ref, ds0_ref, ds1_ref, dd0_ref, dd1_ref, hs_ref, dd_ref):
    row_s = ds0_ref[0, 0] + ds1_ref[0, 0]
    dinv_src = lax.rsqrt(jnp.transpose(row_s, (1, 0)) + 1e-05)
    hs_ref[...] = h_ref[...] * dinv_src
    row_d = dd0_ref[0, 0] + dd1_ref[0, 0]
    dd_ref[...] = lax.rsqrt(jnp.transpose(row_d, (1, 0)) + 1e-05)


def _combine_body(p0_ref, p1_ref, dd_ref, out_ref):
    out_ref[...] = (p0_ref[0] + p1_ref[0]) * dd_ref[...]


@jax.jit
def kernel(x, edge_index, W, b):
    n, d_in = x.shape
    d_out = W.shape[1]
    e = edge_index.shape[1]
    assert e == NC * NS * KBLK * BROWS * CHUNK
    assert n % NS == 0 and n % CHUNK == 0

    n_pad = ((n + NS * 16 - 1) // (NS * 16)) * (NS * 16)
    blk = n_pad // NS
    grid = (NS,)

    edge5d = edge_index.astype(jnp.int32).reshape(2, NC * NS, KBLK, BROWS, CHUNK)

    h = pl.pallas_call(
        _matmul_body,
        grid=grid,
        in_specs=[
            pl.BlockSpec((blk, d_in), lambda i: (i, 0)),
            pl.BlockSpec((d_in, d_out), lambda i: (0, 0)),
            pl.BlockSpec((1, d_out), lambda i: (0, 0)),
        ],
        out_specs=pl.BlockSpec((blk, d_out), lambda i: (i, 0)),
        out_shape=jax.ShapeDtypeStruct((n, d_out), jnp.float32),
    )(x, W, b.reshape(1, d_out))

    dsp, ddp = _degree_kernel(n_pad)(edge5d)

    h_scaled, dinv_dst = pl.pallas_call(
        _scale_body,
        grid=grid,
        in_specs=[
            pl.BlockSpec((blk, d_out), lambda i: (i, 0)),
            pl.BlockSpec((1, 1, 1, blk), lambda i: (0, i, 0, 0)),
            pl.BlockSpec((1, 1, 1, blk), lambda i: (1, i, 0, 0)),
            pl.BlockSpec((1, 1, 1, blk), lambda i: (0, i, 0, 0)),
            pl.BlockSpec((1, 1, 1, blk), lambda i: (1, i, 0, 0)),
        ],
        out_specs=[
            pl.BlockSpec((blk, d_out), lambda i: (i, 0)),
            pl.BlockSpec((blk, 1), lambda i: (i, 0)),
        ],
        out_shape=[
            jax.ShapeDtypeStruct((n, d_out), jnp.float32),
            jax.ShapeDtypeStruct((n_pad, 1), jnp.float32),
        ],
    )(h, dsp, dsp, ddp, ddp)

    partials = _aggregate_kernel(n, d_out)(h_scaled, edge5d)

    cblk = n // 2
    out = pl.pallas_call(
        _combine_body,
        grid=(2,),
        in_specs=[
            pl.BlockSpec((1, cblk, d_out), lambda i: (0, i, 0)),
            pl.BlockSpec((1, cblk, d_out), lambda i: (1, i, 0)),
            pl.BlockSpec((cblk, 1), lambda i: (i, 0)),
        ],
        out_specs=pl.BlockSpec((cblk, d_out), lambda i: (i, 0)),
        out_shape=jax.ShapeDtypeStruct((n, d_out), jnp.float32),
    )(partials, partials, dinv_dst)
    return out


# speedup vs baseline: 36.9361x; 1.0550x over previous
import jax
import jax.numpy as jnp
from jax import lax
from jax.experimental import pallas as pl
from jax.experimental.pallas import tpu as pltpu
from jax.experimental.pallas import tpu_sc as plsc

NC = 2
NS = 16
CHUNK = 80
BROWS = 25
KBLK = 5


def _degree_kernel(n_pad):
    mesh = plsc.VectorSubcoreMesh(core_axis_name="c", subcore_axis_name="s")
    zchunk = n_pad // NS

    def body(edge_hbm, dsp_hbm, ddp_hbm, sidx, didx, ones_v, zbuf, dsh, ddh):
        c = lax.axis_index("c")
        s = lax.axis_index("s")

        def fill_zero(i, _):
            zbuf[pl.ds(i * 16, 16)] = jnp.zeros((16,), jnp.float32)
            return ()
        lax.fori_loop(0, zchunk // 16, fill_zero, ())

        def fill_one(i, _):
            ones_v[pl.ds(i * 16, 16)] = jnp.ones((16,), jnp.float32)
            return ()
        lax.fori_loop(0, CHUNK // 16, fill_one, ())

        pltpu.sync_copy(zbuf, dsh.at[pl.ds(s * zchunk, zchunk)])
        pltpu.sync_copy(zbuf, ddh.at[pl.ds(s * zchunk, zchunk)])
        plsc.subcore_barrier()

        w = c * NS + s

        def block(k, _):
            pltpu.sync_copy(edge_hbm.at[0, w, k], sidx)
            pltpu.sync_copy(edge_hbm.at[1, w, k], didx)

            def step(r, _):
                pltpu.sync_copy(ones_v, dsh.at[sidx.at[r]], add=True)
                pltpu.sync_copy(ones_v, ddh.at[didx.at[r]], add=True)
                return ()
            lax.fori_loop(0, BROWS, step, ())
            return ()
        lax.fori_loop(0, KBLK, block, ())
        plsc.subcore_barrier()

        pltpu.sync_copy(dsh.at[pl.ds(s * zchunk, zchunk)], dsp_hbm.at[c, s, 0])
        pltpu.sync_copy(ddh.at[pl.ds(s * zchunk, zchunk)], ddp_hbm.at[c, s, 0])

    return pl.kernel(
        body,
        out_type=(
            jax.ShapeDtypeStruct((NC, NS, 1, zchunk), jnp.float32),
            jax.ShapeDtypeStruct((NC, NS, 1, zchunk), jnp.float32),
        ),
        mesh=mesh,
        scratch_types=[
            pltpu.VMEM((BROWS, CHUNK), jnp.int32),
            pltpu.VMEM((BROWS, CHUNK), jnp.int32),
            pltpu.VMEM((CHUNK,), jnp.float32),
            pltpu.VMEM((zchunk,), jnp.float32),
            pltpu.VMEM_SHARED((n_pad,), jnp.float32),
            pltpu.VMEM_SHARED((n_pad,), jnp.float32),
        ],
    )


def _aggregate_kernel(n_nodes, d):
    mesh = plsc.VectorSubcoreMesh(core_axis_name="c", subcore_axis_name="s")
    zchunks = n_nodes // CHUNK

    def body(h_hbm, edge_hbm, out_hbm, sidx, didx, bufs, acc, gsems, ssems):
        c = lax.axis_index("c")
        s = lax.axis_index("s")
        dg = d // 16

        def fill_zero(i, _):
            bufs[0, i // dg, pl.ds((i % dg) * 16, 16)] = jnp.zeros((16,), jnp.float32)
            return ()
        lax.fori_loop(0, CHUNK * dg, fill_zero, ())

        def zstep(i, _):
            j = s + i * NS

            @pl.when(j < zchunks)
            def _():
                pltpu.sync_copy(bufs.at[0], acc.at[pl.ds(j * CHUNK, CHUNK)])
            return ()
        lax.fori_loop(0, (zchunks + NS - 1) // NS, zstep, ())
        plsc.subcore_barrier()

        w = c * NS + s

        for k in range(KBLK):
            pltpu.sync_copy(edge_hbm.at[0, w, k], sidx)
            pltpu.sync_copy(edge_hbm.at[1, w, k], didx)
            for r0 in range(3):
                pltpu.async_copy(h_hbm.at[sidx.at[r0]], bufs.at[r0], gsems.at[r0])

            for r in range(BROWS):
                bi = r % 4
                pltpu.make_async_copy(
                    h_hbm.at[sidx.at[r]], bufs.at[bi], gsems.at[bi]).wait()
                pltpu.async_copy(bufs.at[bi], acc.at[didx.at[r]], ssems.at[bi],
                                 add=True)
                nr = r + 3
                if nr < BROWS:
                    nbi = nr % 4
                    if nr >= 4:
                        pltpu.make_async_copy(
                            bufs.at[nbi], acc.at[didx.at[nr - 4]],
                            ssems.at[nbi]).wait()
                    pltpu.async_copy(h_hbm.at[sidx.at[nr]], bufs.at[nbi],
                                     gsems.at[nbi])
            for rr in range(BROWS - 4, BROWS):
                bi = rr % 4
                pltpu.make_async_copy(
                    bufs.at[bi], acc.at[didx.at[rr]], ssems.at[bi]).wait()
        plsc.subcore_barrier()

        def wstep(i, _):
            j = s + i * NS

            @pl.when(j < zchunks)
            def _():
                pltpu.sync_copy(acc.at[pl.ds(j * CHUNK, CHUNK)],
                                out_hbm.at[c, pl.ds(j * CHUNK, CHUNK)])
            return ()
        lax.fori_loop(0, (zchunks + NS - 1) // NS, wstep, ())

    return pl.kernel(
        body,
        out_type=jax.ShapeDtypeStruct((NC, n_nodes, d), jnp.float32),
        mesh=mesh,
        scratch_types=[
            pltpu.VMEM((BROWS, CHUNK), jnp.int32),
            pltpu.VMEM((BROWS, CHUNK), jnp.int32),
            pltpu.VMEM((4, CHUNK, d), jnp.float32),
            pltpu.VMEM_SHARED((n_nodes, d), jnp.float32),
            pltpu.SemaphoreType.DMA((4,)),
            pltpu.SemaphoreType.DMA((4,)),
        ],
    )


def _matmul_body(x_ref, w_ref, b_ref, h_ref):
    h_ref[...] = jnp.dot(x_ref[...], w_ref[...],
                         preferred_element_type=jnp.float32) + b_ref[...]


def _scale_body(h_ref, ds0_ref, ds1_ref, dd0_ref, dd1_ref, hs_ref, dd_ref):
    row_s = ds0_ref[0, 0] + ds1_ref[0, 0]
    dinv_src = lax.rsqrt(jnp.transpose(row_s, (1, 0)) + 1e-05)
    hs_ref[...] = h_ref[...] * dinv_src
    row_d = dd0_ref[0, 0] + dd1_ref[0, 0]
    dd_ref[...] = lax.rsqrt(jnp.transpose(row_d, (1, 0)) + 1e-05)


def _combine_body(p0_ref, p1_ref, dd_ref, out_ref):
    out_ref[...] = (p0_ref[0] + p1_ref[0]) * dd_ref[...]


@jax.jit
def kernel(x, edge_index, W, b):
    n, d_in = x.shape
    d_out = W.shape[1]
    e = edge_index.shape[1]
    assert e == NC * NS * KBLK * BROWS * CHUNK
    assert n % NS == 0 and n % CHUNK == 0

    n_pad = ((n + NS * 16 - 1) // (NS * 16)) * (NS * 16)
    blk = n_pad // NS
    grid = (NS,)

    edge5d = edge_index.astype(jnp.int32).reshape(2, NC * NS, KBLK, BROWS, CHUNK)

    h = pl.pallas_call(
        _matmul_body,
        grid=grid,
        in_specs=[
            pl.BlockSpec((blk, d_in), lambda i: (i, 0)),
            pl.BlockSpec((d_in, d_out), lambda i: (0, 0)),
            pl.BlockSpec((1, d_out), lambda i: (0, 0)),
        ],
        out_specs=pl.BlockSpec((blk, d_out), lambda i: (i, 0)),
        out_shape=jax.ShapeDtypeStruct((n, d_out), jnp.float32),
    )(x, W, b.reshape(1, d_out))

    dsp, ddp = _degree_kernel(n_pad)(edge5d)

    h_scaled, dinv_dst = pl.pallas_call(
        _scale_body,
        grid=grid,
        in_specs=[
            pl.BlockSpec((blk, d_out), lambda i: (i, 0)),
            pl.BlockSpec((1, 1, 1, blk), lambda i: (0, i, 0, 0)),
            pl.BlockSpec((1, 1, 1, blk), lambda i: (1, i, 0, 0)),
            pl.BlockSpec((1, 1, 1, blk), lambda i: (0, i, 0, 0)),
            pl.BlockSpec((1, 1, 1, blk), lambda i: (1, i, 0, 0)),
        ],
        out_specs=[
            pl.BlockSpec((blk, d_out), lambda i: (i, 0)),
            pl.BlockSpec((blk, 1), lambda i: (i, 0)),
        ],
        out_shape=[
            jax.ShapeDtypeStruct((n, d_out), jnp.float32),
            jax.ShapeDtypeStruct((n_pad, 1), jnp.float32),
        ],
    )(h, dsp, dsp, ddp, ddp)

    partials = _aggregate_kernel(n, d_out)(h_scaled, edge5d)

    cblk = n // 2
    out = pl.pallas_call(
        _combine_body,
        grid=(2,),
        in_specs=[
            pl.BlockSpec((1, cblk, d_out), lambda i: (0, i, 0)),
            pl.BlockSpec((1, cblk, d_out), lambda i: (1, i, 0)),
            pl.BlockSpec((cblk, 1), lambda i: (i, 0)),
        ],
        out_specs=pl.BlockSpec((cblk, d_out), lambda i: (i, 0)),
        out_shape=jax.ShapeDtypeStruct((n, d_out), jnp.float32),
    )(partials, partials, dinv_dst)
    return out


# speedup vs baseline: 37.7627x; 1.0224x over previous
import jax
import jax.numpy as jnp
from jax import lax
from jax.experimental import pallas as pl
from jax.experimental.pallas import tpu as pltpu
from jax.experimental.pallas import tpu_sc as plsc

NC = 2
NS = 16
CHUNK = 80
BROWS = 25
KBLK = 5


def _degree_kernel(n_pad):
    mesh = plsc.VectorSubcoreMesh(core_axis_name="c", subcore_axis_name="s")
    zchunk = n_pad // NS

    def body(edge_hbm, dsp_hbm, ddp_hbm, sidx, didx, ones_v, zbuf, dsh, ddh):
        c = lax.axis_index("c")
        s = lax.axis_index("s")

        def fill_zero(i, _):
            zbuf[pl.ds(i * 16, 16)] = jnp.zeros((16,), jnp.float32)
            return ()
        lax.fori_loop(0, zchunk // 16, fill_zero, ())

        def fill_one(i, _):
            ones_v[pl.ds(i * 16, 16)] = jnp.ones((16,), jnp.float32)
            return ()
        lax.fori_loop(0, CHUNK // 16, fill_one, ())

        pltpu.sync_copy(zbuf, dsh.at[pl.ds(s * zchunk, zchunk)])
        pltpu.sync_copy(zbuf, ddh.at[pl.ds(s * zchunk, zchunk)])
        plsc.subcore_barrier()

        w = c * NS + s
        pltpu.sync_copy(edge_hbm.at[0, w], sidx)
        pltpu.sync_copy(edge_hbm.at[1, w], didx)

        def step(g, _):
            kb = g // BROWS
            r = g % BROWS
            pltpu.sync_copy(ones_v, dsh.at[sidx.at[kb, r]], add=True)
            pltpu.sync_copy(ones_v, ddh.at[didx.at[kb, r]], add=True)
            return ()
        lax.fori_loop(0, KBLK * BROWS, step, ())
        plsc.subcore_barrier()

        pltpu.sync_copy(dsh.at[pl.ds(s * zchunk, zchunk)], dsp_hbm.at[c, s, 0])
        pltpu.sync_copy(ddh.at[pl.ds(s * zchunk, zchunk)], ddp_hbm.at[c, s, 0])

    return pl.kernel(
        body,
        out_type=(
            jax.ShapeDtypeStruct((NC, NS, 1, zchunk), jnp.float32),
            jax.ShapeDtypeStruct((NC, NS, 1, zchunk), jnp.float32),
        ),
        mesh=mesh,
        scratch_types=[
            pltpu.VMEM((KBLK, BROWS, CHUNK), jnp.int32),
            pltpu.VMEM((KBLK, BROWS, CHUNK), jnp.int32),
            pltpu.VMEM((CHUNK,), jnp.float32),
            pltpu.VMEM((zchunk,), jnp.float32),
            pltpu.VMEM_SHARED((n_pad,), jnp.float32),
            pltpu.VMEM_SHARED((n_pad,), jnp.float32),
        ],
    )


def _aggregate_kernel(n_nodes, d):
    mesh = plsc.VectorSubcoreMesh(core_axis_name="c", subcore_axis_name="s")
    zchunks = n_nodes // CHUNK

    def body(h_hbm, edge_hbm, out_hbm, sidx, didx, bufs, acc, gsems, ssems):
        c = lax.axis_index("c")
        s = lax.axis_index("s")
        dg = d // 16

        def fill_zero(i, _):
            bufs[0, i // dg, pl.ds((i % dg) * 16, 16)] = jnp.zeros((16,), jnp.float32)
            return ()
        lax.fori_loop(0, CHUNK * dg, fill_zero, ())

        def zstep(i, _):
            j = s + i * NS

            @pl.when(j < zchunks)
            def _():
                pltpu.sync_copy(bufs.at[0], acc.at[pl.ds(j * CHUNK, CHUNK)])
            return ()
        lax.fori_loop(0, (zchunks + NS - 1) // NS, zstep, ())
        plsc.subcore_barrier()

        w = c * NS + s

        for k in range(KBLK):
            pltpu.sync_copy(edge_hbm.at[0, w, k], sidx)
            pltpu.sync_copy(edge_hbm.at[1, w, k], didx)
            for r0 in range(3):
                pltpu.async_copy(h_hbm.at[sidx.at[r0]], bufs.at[r0], gsems.at[r0])

            for r in range(BROWS):
                bi = r % 4
                pltpu.make_async_copy(
                    h_hbm.at[sidx.at[r]], bufs.at[bi], gsems.at[bi]).wait()
                pltpu.async_copy(bufs.at[bi], acc.at[didx.at[r]], ssems.at[bi],
                                 add=True)
                nr = r + 3
                if nr < BROWS:
                    nbi = nr % 4
                    if nr >= 4:
                        pltpu.make_async_copy(
                            bufs.at[nbi], acc.at[didx.at[nr - 4]],
                            ssems.at[nbi]).wait()
                    pltpu.async_copy(h_hbm.at[sidx.at[nr]], bufs.at[nbi],
                                     gsems.at[nbi])
            for rr in range(BROWS - 4, BROWS):
                bi = rr % 4
                pltpu.make_async_copy(
                    bufs.at[bi], acc.at[didx.at[rr]], ssems.at[bi]).wait()
        plsc.subcore_barrier()

        def wstep(i, _):
            j = s + i * NS

            @pl.when(j < zchunks)
            def _():
                pltpu.sync_copy(acc.at[pl.ds(j * CHUNK, CHUNK)],
                                out_hbm.at[c, pl.ds(j * CHUNK, CHUNK)])
            return ()
        lax.fori_loop(0, (zchunks + NS - 1) // NS, wstep, ())

    return pl.kernel(
        body,
        out_type=jax.ShapeDtypeStruct((NC, n_nodes, d), jnp.float32),
        mesh=mesh,
        scratch_types=[
            pltpu.VMEM((BROWS, CHUNK), jnp.int32),
            pltpu.VMEM((BROWS, CHUNK), jnp.int32),
            pltpu.VMEM((4, CHUNK, d), jnp.float32),
            pltpu.VMEM_SHARED((n_nodes, d), jnp.float32),
            pltpu.SemaphoreType.DMA((4,)),
            pltpu.SemaphoreType.DMA((4,)),
        ],
    )


def _matmul_body(x_ref, w_ref, b_ref, h_ref):
    h_ref[...] = jnp.dot(x_ref[...], w_ref[...],
                         preferred_element_type=jnp.float32) + b_ref[...]


def _scale_body(h_ref, ds0_ref, ds1_ref, dd0_ref, dd1_ref, hs_ref, dd_ref):
    row_s = ds0_ref[0, 0] + ds1_ref[0, 0]
    dinv_src = lax.rsqrt(jnp.transpose(row_s, (1, 0)) + 1e-05)
    hs_ref[...] = h_ref[...] * dinv_src
    row_d = dd0_ref[0, 0] + dd1_ref[0, 0]
    dd_ref[...] = lax.rsqrt(jnp.transpose(row_d, (1, 0)) + 1e-05)


def _combine_body(p0_ref, p1_ref, dd_ref, out_ref):
    out_ref[...] = (p0_ref[0] + p1_ref[0]) * dd_ref[...]


@jax.jit
def kernel(x, edge_index, W, b):
    n, d_in = x.shape
    d_out = W.shape[1]
    e = edge_index.shape[1]
    assert e == NC * NS * KBLK * BROWS * CHUNK
    assert n % NS == 0 and n % CHUNK == 0

    n_pad = ((n + NS * 16 - 1) // (NS * 16)) * (NS * 16)
    blk = n_pad // NS
    grid = (NS,)

    edge5d = edge_index.astype(jnp.int32).reshape(2, NC * NS, KBLK, BROWS, CHUNK)

    h = pl.pallas_call(
        _matmul_body,
        grid=grid,
        in_specs=[
            pl.BlockSpec((blk, d_in), lambda i: (i, 0)),
            pl.BlockSpec((d_in, d_out), lambda i: (0, 0)),
            pl.BlockSpec((1, d_out), lambda i: (0, 0)),
        ],
        out_specs=pl.BlockSpec((blk, d_out), lambda i: (i, 0)),
        out_shape=jax.ShapeDtypeStruct((n, d_out), jnp.float32),
    )(x, W, b.reshape(1, d_out))

    dsp, ddp = _degree_kernel(n_pad)(edge5d)

    h_scaled, dinv_dst = pl.pallas_call(
        _scale_body,
        grid=grid,
        in_specs=[
            pl.BlockSpec((blk, d_out), lambda i: (i, 0)),
            pl.BlockSpec((1, 1, 1, blk), lambda i: (0, i, 0, 0)),
            pl.BlockSpec((1, 1, 1, blk), lambda i: (1, i, 0, 0)),
            pl.BlockSpec((1, 1, 1, blk), lambda i: (0, i, 0, 0)),
            pl.BlockSpec((1, 1, 1, blk), lambda i: (1, i, 0, 0)),
        ],
        out_specs=[
            pl.BlockSpec((blk, d_out), lambda i: (i, 0)),
            pl.BlockSpec((blk, 1), lambda i: (i, 0)),
        ],
        out_shape=[
            jax.ShapeDtypeStruct((n, d_out), jnp.float32),
            jax.ShapeDtypeStruct((n_pad, 1), jnp.float32),
        ],
    )(h, dsp, dsp, ddp, ddp)

    partials = _aggregate_kernel(n, d_out)(h_scaled, edge5d)

    cblk = n // 2
    out = pl.pallas_call(
        _combine_body,
        grid=(2,),
        in_specs=[
            pl.BlockSpec((1, cblk, d_out), lambda i: (0, i, 0)),
            pl.BlockSpec((1, cblk, d_out), lambda i: (1, i, 0)),
            pl.BlockSpec((cblk, 1), lambda i: (i, 0)),
        ],
        out_specs=pl.BlockSpec((cblk, d_out), lambda i: (i, 0)),
        out_shape=jax.ShapeDtypeStruct((n, d_out), jnp.float32),
    )(partials, partials, dinv_dst)
    return out
